# Initial kernel scaffold; baseline (speedup 1.0000x reference)
import functools

import jax
import jax.numpy as jnp
from jax import lax
from jax.experimental import pallas as pl
from jax.experimental.pallas import tpu as pltpu
from jax.experimental.pallas import tpu_sc as plsc

N = 10000
E = 320000
D = 256
HALF = 128
NC = 2
NS = 16
EPT = E // NS
CHUNK = 125
NCHUNK = EPT // CHUNK
RPT = N // NS

_MESH = plsc.VectorSubcoreMesh(core_axis_name="c", subcore_axis_name="s")


def _sc_agg_build(with_cnt):
  def body(xblk, srcp, dst2, zblk, zcnt, *rest):
    if with_cnt:
      agg, cnt, src_idx, dst_idx, gbuf, ones_buf, acc, cnt_sp = rest
    else:
      agg, src_idx, dst_idx, gbuf, ones_buf, acc, cnt_sp = rest
    c = lax.axis_index("c")
    s = lax.axis_index("s")
    w = c * NS + s

    pltpu.sync_copy(srcp.at[w], src_idx)
    pltpu.sync_copy(dst2.at[s], dst_idx)

    rows = pl.ds(s * RPT, RPT)
    pltpu.sync_copy(zblk.at[rows], acc.at[rows])
    if with_cnt:
      @pl.when(c == 0)
      def _():
        pltpu.sync_copy(zcnt.at[rows], cnt_sp.at[rows])

        def fill(i, carry):
          ones_buf[i] = jnp.ones((16,), jnp.float32)
          return carry
        lax.fori_loop(0, CHUNK, fill, 0)

    plsc.subcore_barrier()

    def chunk(j, carry):
      pltpu.sync_copy(xblk.at[src_idx.at[j]], gbuf)
      pltpu.sync_copy(gbuf, acc.at[dst_idx.at[j]], add=True)
      if with_cnt:
        @pl.when(c == 0)
        def _():
          pltpu.sync_copy(ones_buf, cnt_sp.at[dst_idx.at[j]], add=True)
      return carry
    lax.fori_loop(0, NCHUNK, chunk, 0)

    plsc.subcore_barrier()

    out_rows = pl.ds(c * N + s * RPT, RPT)
    pltpu.sync_copy(acc.at[rows], agg.at[out_rows])
    if with_cnt:
      @pl.when(c == 0)
      def _():
        pltpu.sync_copy(cnt_sp.at[rows], cnt.at[rows])

  if with_cnt:
    out_type = (jax.ShapeDtypeStruct((2 * N, HALF), jnp.float32),
                jax.ShapeDtypeStruct((N, 16), jnp.float32))
  else:
    out_type = jax.ShapeDtypeStruct((2 * N, HALF), jnp.float32)
  return pl.kernel(
      body,
      out_type=out_type,
      mesh=_MESH,
      scratch_types=[
          pltpu.VMEM((NCHUNK, CHUNK), jnp.int32),
          pltpu.VMEM((NCHUNK, CHUNK), jnp.int32),
          pltpu.VMEM((CHUNK, HALF), jnp.float32),
          pltpu.VMEM((CHUNK, 16), jnp.float32),
          pltpu.VMEM_SHARED((N, HALF), jnp.float32),
          pltpu.VMEM_SHARED((N, 16), jnp.float32),
      ],
  )


_sc_agg_cnt = _sc_agg_build(True)
_sc_agg = _sc_agg_build(False)

R = 1000
NRT = N // R


def _dense_build(do_relu, blocked_out):
  def body(agga, aggb, xa, xb, cnt, wl, wr, bias, out):
    inv = 1.0 / jnp.maximum(cnt[:, 0:1], 1.0)
    wl_ = wl[...]
    wr_ = wr[...]
    ma = agga[...] * inv
    mb = aggb[...] * inv
    r = (
        jnp.dot(ma, wl_[:HALF], preferred_element_type=jnp.float32)
        + jnp.dot(mb, wl_[HALF:], preferred_element_type=jnp.float32)
        + jnp.dot(xa[...], wr_[:HALF], preferred_element_type=jnp.float32)
        + jnp.dot(xb[...], wr_[HALF:], preferred_element_type=jnp.float32)
        + bias[...]
    )
    if do_relu:
      r = jnp.maximum(r, 0.0)
    out[...] = r

  if blocked_out:
    out_sds = jax.ShapeDtypeStruct((2 * N, HALF), jnp.float32)
    out_spec = pl.BlockSpec((R, HALF), lambda i, c: (c * NRT + i, 0))
  else:
    out_sds = jax.ShapeDtypeStruct((N, D), jnp.float32)
    out_spec = pl.BlockSpec((R, HALF), lambda i, c: (i, c))
  return pl.pallas_call(
      body,
      grid=(NRT, NC),
      in_specs=[
          pl.BlockSpec((R, HALF), lambda i, c: (i, 0)),
          pl.BlockSpec((R, HALF), lambda i, c: (NRT + i, 0)),
          pl.BlockSpec((R, HALF), lambda i, c: (i, 0)),
          pl.BlockSpec((R, HALF), lambda i, c: (NRT + i, 0)),
          pl.BlockSpec((R, 16), lambda i, c: (i, 0)),
          pl.BlockSpec((D, HALF), lambda i, c: (0, c)),
          pl.BlockSpec((D, HALF), lambda i, c: (0, c)),
          pl.BlockSpec((1, HALF), lambda i, c: (c, 0)),
      ],
      out_specs=out_spec,
      out_shape=out_sds,
  )


_dense_relu = _dense_build(True, True)
_dense_final = _dense_build(False, False)


@jax.jit
def kernel(x, edge_index, W1l, W1r, b1, W2l, W2r, b2):
  src = edge_index[0].astype(jnp.int32)
  dst = edge_index[1].astype(jnp.int32)
  srcp = jnp.stack([src, src + N]).reshape(NC * NS, NCHUNK, CHUNK)
  dst2 = dst.reshape(NS, NCHUNK, CHUNK)
  xblk = x.reshape(N, NC, HALF).transpose(1, 0, 2).reshape(NC * N, HALF)
  zblk = jnp.zeros((N, HALF), jnp.float32)
  zcnt = jnp.zeros((N, 16), jnp.float32)
  b1b = b1.reshape(NC, HALF)
  b2b = b2.reshape(NC, HALF)

  agg, cnt = _sc_agg_cnt(xblk, srcp, dst2, zblk, zcnt)
  h1 = _dense_relu(agg, agg, xblk, xblk, cnt, W1l, W1r, b1b)
  agg = _sc_agg(h1, srcp, dst2, zblk, zcnt)
  h2 = _dense_relu(agg, agg, h1, h1, cnt, W2l, W2r, b2b)
  agg = _sc_agg(h2, srcp, dst2, zblk, zcnt)
  return _dense_final(agg, agg, h2, h2, cnt, W2l, W2r, b2b)


# speedup vs baseline: 2.5190x; 2.5190x over previous
import jax
import jax.numpy as jnp
from jax import lax
from jax.experimental import pallas as pl
from jax.experimental.pallas import tpu as pltpu
from jax.experimental.pallas import tpu_sc as plsc

N = 10000
E = 320000
D = 256
HALF = 128
NC = 2
NS = 16
CHUNK = 128
NCHUNK = 160
CCHUNK = 80
CPT = NCHUNK * CHUNK
EP = NS * CPT
EPAD = EP - E
ACCN = N + 8
WA = 640
WI = ACCN - (NS - 1) * WA
WO = N - (NS - 1) * WA

_MESH = plsc.VectorSubcoreMesh(core_axis_name="c", subcore_axis_name="s")


def _agg_body(xblk, srcp, dstp, zblk, agg, src_v, dst_v, gbuf, acc, sem):
  c = lax.axis_index("c")
  s = lax.axis_index("s")
  start = pl.multiple_of(s * WA, 8)

  @pl.when(s < NS - 1)
  def _():
    pltpu.sync_copy(zblk, acc.at[pl.ds(start, WA)])

  @pl.when(s == NS - 1)
  def _():
    pltpu.sync_copy(zblk.at[pl.ds(0, WI)], acc.at[pl.ds(start, WI)])

  plsc.subcore_barrier()

  src_base = (c * NS + s) * CPT
  dst_base = s * CPT

  def chunk(j, carry):
    off = pl.multiple_of(j * CHUNK, 8)
    pltpu.sync_copy(srcp.at[pl.ds(src_base + off, CHUNK)], src_v)
    pltpu.sync_copy(dstp.at[pl.ds(dst_base + off, CHUNK)], dst_v)
    pltpu.async_copy(xblk.at[src_v], gbuf, sem).wait()
    pltpu.sync_copy(gbuf, acc.at[dst_v], add=True)
    return carry

  lax.fori_loop(0, NCHUNK, chunk, 0)

  plsc.subcore_barrier()

  out_start = pl.multiple_of(c * N + s * WA, 8)

  @pl.when(s < NS - 1)
  def _():
    pltpu.sync_copy(acc.at[pl.ds(start, WA)], agg.at[pl.ds(out_start, WA)])

  @pl.when(s == NS - 1)
  def _():
    pltpu.sync_copy(acc.at[pl.ds(start, WO)], agg.at[pl.ds(out_start, WO)])


_sc_agg = pl.kernel(
    _agg_body,
    out_type=jax.ShapeDtypeStruct((NC * N, HALF), jnp.float32),
    mesh=_MESH,
    scratch_types=[
        pltpu.VMEM((CHUNK,), jnp.int32),
        pltpu.VMEM((CHUNK,), jnp.int32),
        pltpu.VMEM((CHUNK, HALF), jnp.float32),
        pltpu.VMEM_SHARED((ACCN, HALF), jnp.float32),
        pltpu.SemaphoreType.DMA,
    ],
)


def _cnt_body(dstp, zblk, onesw, cntb, dst_v, ones_v, acc):
  c = lax.axis_index("c")
  s = lax.axis_index("s")
  start = pl.multiple_of(s * WA, 8)

  @pl.when(s < NS - 1)
  def _():
    pltpu.sync_copy(zblk, acc.at[pl.ds(start, WA)])

  @pl.when(s == NS - 1)
  def _():
    pltpu.sync_copy(zblk.at[pl.ds(0, WI)], acc.at[pl.ds(start, WI)])

  pltpu.sync_copy(onesw, ones_v)
  plsc.subcore_barrier()

  dst_base = (c * NS + s) * CCHUNK * CHUNK

  def chunk(j, carry):
    off = pl.multiple_of(j * CHUNK, 8)
    pltpu.sync_copy(dstp.at[pl.ds(dst_base + off, CHUNK)], dst_v)
    pltpu.sync_copy(ones_v, acc.at[dst_v], add=True)
    return carry

  lax.fori_loop(0, CCHUNK, chunk, 0)

  plsc.subcore_barrier()

  out_start = pl.multiple_of(c * N + s * WA, 8)

  @pl.when(s < NS - 1)
  def _():
    pltpu.sync_copy(acc.at[pl.ds(start, WA)], cntb.at[pl.ds(out_start, WA)])

  @pl.when(s == NS - 1)
  def _():
    pltpu.sync_copy(acc.at[pl.ds(start, WO)], cntb.at[pl.ds(out_start, WO)])


_sc_cnt = pl.kernel(
    _cnt_body,
    out_type=jax.ShapeDtypeStruct((NC * N, HALF), jnp.float32),
    mesh=_MESH,
    scratch_types=[
        pltpu.VMEM((CHUNK,), jnp.int32),
        pltpu.VMEM((CHUNK, HALF), jnp.float32),
        pltpu.VMEM_SHARED((ACCN, HALF), jnp.float32),
    ],
)

R = 1000
NRT = N // R


def _dense_build(do_relu, blocked_out):
  def body(agga, aggb, xa, xb, cnta, cntb, wl, wr, bias, out):
    inv = 1.0 / jnp.maximum(cnta[:, 0:1] + cntb[:, 0:1], 1.0)
    wl_ = wl[...]
    wr_ = wr[...]
    ma = agga[...] * inv
    mb = aggb[...] * inv
    r = (
        jnp.dot(ma, wl_[:HALF], preferred_element_type=jnp.float32)
        + jnp.dot(mb, wl_[HALF:], preferred_element_type=jnp.float32)
        + jnp.dot(xa[...], wr_[:HALF], preferred_element_type=jnp.float32)
        + jnp.dot(xb[...], wr_[HALF:], preferred_element_type=jnp.float32)
        + bias[0]
    )
    if do_relu:
      r = jnp.maximum(r, 0.0)
    out[...] = r

  if blocked_out:
    out_sds = jax.ShapeDtypeStruct((NC * N, HALF), jnp.float32)
    out_spec = pl.BlockSpec((R, HALF), lambda i, c: (c * NRT + i, 0))
  else:
    out_sds = jax.ShapeDtypeStruct((N, D), jnp.float32)
    out_spec = pl.BlockSpec((R, HALF), lambda i, c: (i, c))
  return pl.pallas_call(
      body,
      grid=(NRT, NC),
      in_specs=[
          pl.BlockSpec((R, HALF), lambda i, c: (i, 0)),
          pl.BlockSpec((R, HALF), lambda i, c: (NRT + i, 0)),
          pl.BlockSpec((R, HALF), lambda i, c: (i, 0)),
          pl.BlockSpec((R, HALF), lambda i, c: (NRT + i, 0)),
          pl.BlockSpec((R, HALF), lambda i, c: (i, 0)),
          pl.BlockSpec((R, HALF), lambda i, c: (NRT + i, 0)),
          pl.BlockSpec((D, HALF), lambda i, c: (0, c)),
          pl.BlockSpec((D, HALF), lambda i, c: (0, c)),
          pl.BlockSpec((1, 1, HALF), lambda i, c: (c, 0, 0)),
      ],
      out_specs=out_spec,
      out_shape=out_sds,
  )


_dense_relu = _dense_build(True, True)
_dense_final = _dense_build(False, False)


@jax.jit
def kernel(x, edge_index, W1l, W1r, b1, W2l, W2r, b2):
  src = edge_index[0].astype(jnp.int32)
  dst = edge_index[1].astype(jnp.int32)
  srcf = jnp.concatenate([src, jnp.zeros((EPAD,), jnp.int32)])
  srcp = jnp.concatenate([srcf, srcf + N])
  dstp = jnp.concatenate([dst, jnp.full((EPAD,), N, jnp.int32)])
  xblk = x.reshape(N, NC, HALF).transpose(1, 0, 2).reshape(NC * N, HALF)
  zblk = jnp.zeros((WA, HALF), jnp.float32)
  onesw = jnp.ones((CHUNK, HALF), jnp.float32)
  b1b = b1.reshape(NC, 1, HALF)
  b2b = b2.reshape(NC, 1, HALF)

  cnt = _sc_cnt(dstp, zblk, onesw)
  agg = _sc_agg(xblk, srcp, dstp, zblk)
  h1 = _dense_relu(agg, agg, xblk, xblk, cnt, cnt, W1l, W1r, b1b)
  agg = _sc_agg(h1, srcp, dstp, zblk)
  h2 = _dense_relu(agg, agg, h1, h1, cnt, cnt, W2l, W2r, b2b)
  agg = _sc_agg(h2, srcp, dstp, zblk)
  return _dense_final(agg, agg, h2, h2, cnt, cnt, W2l, W2r, b2b)


# speedup vs baseline: 3.0617x; 1.2154x over previous
import jax
import jax.numpy as jnp
from jax import lax
from jax.experimental import pallas as pl
from jax.experimental.pallas import tpu as pltpu
from jax.experimental.pallas import tpu_sc as plsc

N = 10000
E = 320000
D = 256
HALF = 128
NC = 2
NS = 16
CHUNK = 128
NBUF = 2
SG = 16
NCHUNK = 160
NSG = NCHUNK // SG
CCHUNK = 80
CPT = NCHUNK * CHUNK
EP = NS * CPT
EPAD = EP - E
ACCN = N + 8
WA = 640
WI = ACCN - (NS - 1) * WA
WO = N - (NS - 1) * WA

_MESH = plsc.VectorSubcoreMesh(core_axis_name="c", subcore_axis_name="s")


def _agg_body(xblk, srcp, dstp, zblk, agg, src_all, dst_all,
              g0, g1, acc, s0, s1):
  c = lax.axis_index("c")
  s = lax.axis_index("s")
  start = pl.multiple_of(s * WA, 8)

  @pl.when(s < NS - 1)
  def _():
    pltpu.sync_copy(zblk, acc.at[pl.ds(start, WA)])

  @pl.when(s == NS - 1)
  def _():
    pltpu.sync_copy(zblk.at[pl.ds(0, WI)], acc.at[pl.ds(start, WI)])

  plsc.subcore_barrier()

  src_base = (c * NS + s) * CPT
  dst_base = s * CPT

  gbufs = (g0, g1)
  sems = (s0, s1)

  def supergroup(jsg, carry):
    sg_off = pl.multiple_of(jsg * (SG * CHUNK), 128)
    pltpu.sync_copy(srcp.at[pl.ds(src_base + sg_off, SG * CHUNK)], src_all)
    pltpu.sync_copy(dstp.at[pl.ds(dst_base + sg_off, SG * CHUNK)], dst_all)

    def group(jg, carry2):
      base = pl.multiple_of(jg * (NBUF * CHUNK), 128)
      cps = []
      for b in range(NBUF):
        off = pl.multiple_of(base + b * CHUNK, 128)
        cps.append(
            pltpu.async_copy(
                xblk.at[src_all.at[pl.ds(off, CHUNK)]], gbufs[b], sems[b]))
      for b in range(NBUF):
        off = pl.multiple_of(base + b * CHUNK, 128)
        cps[b].wait()
        pltpu.sync_copy(
            gbufs[b], acc.at[dst_all.at[pl.ds(off, CHUNK)]], add=True)
      return carry2

    lax.fori_loop(0, SG // NBUF, group, 0)
    return carry

  lax.fori_loop(0, NSG, supergroup, 0)

  plsc.subcore_barrier()

  out_start = pl.multiple_of(c * N + s * WA, 8)

  @pl.when(s < NS - 1)
  def _():
    pltpu.sync_copy(acc.at[pl.ds(start, WA)], agg.at[pl.ds(out_start, WA)])

  @pl.when(s == NS - 1)
  def _():
    pltpu.sync_copy(acc.at[pl.ds(start, WO)], agg.at[pl.ds(out_start, WO)])


_sc_agg = pl.kernel(
    _agg_body,
    out_type=jax.ShapeDtypeStruct((NC * N, HALF), jnp.float32),
    mesh=_MESH,
    scratch_types=[
        pltpu.VMEM((SG * CHUNK,), jnp.int32),
        pltpu.VMEM((SG * CHUNK,), jnp.int32),
        pltpu.VMEM((CHUNK, HALF), jnp.float32),
        pltpu.VMEM((CHUNK, HALF), jnp.float32),
        pltpu.VMEM_SHARED((ACCN, HALF), jnp.float32),
        pltpu.SemaphoreType.DMA,
        pltpu.SemaphoreType.DMA,
    ],
)


def _cnt_body(dstp, zblk, onesw, cntb, dst_v, ones_v, acc):
  c = lax.axis_index("c")
  s = lax.axis_index("s")
  start = pl.multiple_of(s * WA, 8)

  @pl.when(s < NS - 1)
  def _():
    pltpu.sync_copy(zblk, acc.at[pl.ds(start, WA)])

  @pl.when(s == NS - 1)
  def _():
    pltpu.sync_copy(zblk.at[pl.ds(0, WI)], acc.at[pl.ds(start, WI)])

  pltpu.sync_copy(onesw, ones_v)
  plsc.subcore_barrier()

  dst_base = (c * NS + s) * CCHUNK * CHUNK

  def chunk(j, carry):
    off = pl.multiple_of(j * CHUNK, 8)
    pltpu.sync_copy(dstp.at[pl.ds(dst_base + off, CHUNK)], dst_v)
    pltpu.sync_copy(ones_v, acc.at[dst_v], add=True)
    return carry

  lax.fori_loop(0, CCHUNK, chunk, 0)

  plsc.subcore_barrier()

  out_start = pl.multiple_of(c * N + s * WA, 8)

  @pl.when(s < NS - 1)
  def _():
    pltpu.sync_copy(acc.at[pl.ds(start, WA)], cntb.at[pl.ds(out_start, WA)])

  @pl.when(s == NS - 1)
  def _():
    pltpu.sync_copy(acc.at[pl.ds(start, WO)], cntb.at[pl.ds(out_start, WO)])


_sc_cnt = pl.kernel(
    _cnt_body,
    out_type=jax.ShapeDtypeStruct((NC * N, HALF), jnp.float32),
    mesh=_MESH,
    scratch_types=[
        pltpu.VMEM((CHUNK,), jnp.int32),
        pltpu.VMEM((CHUNK, HALF), jnp.float32),
        pltpu.VMEM_SHARED((ACCN, HALF), jnp.float32),
    ],
)

R = 1000
NRT = N // R


def _dense_build(do_relu, blocked_out):
  def body(agga, aggb, xa, xb, cnta, cntb, wl, wr, bias, out):
    inv = 1.0 / jnp.maximum(cnta[:, 0:1] + cntb[:, 0:1], 1.0)
    wl_ = wl[...]
    wr_ = wr[...]
    ma = agga[...] * inv
    mb = aggb[...] * inv
    r = (
        jnp.dot(ma, wl_[:HALF], preferred_element_type=jnp.float32)
        + jnp.dot(mb, wl_[HALF:], preferred_element_type=jnp.float32)
        + jnp.dot(xa[...], wr_[:HALF], preferred_element_type=jnp.float32)
        + jnp.dot(xb[...], wr_[HALF:], preferred_element_type=jnp.float32)
        + bias[0]
    )
    if do_relu:
      r = jnp.maximum(r, 0.0)
    out[...] = r

  if blocked_out:
    out_sds = jax.ShapeDtypeStruct((NC * N, HALF), jnp.float32)
    out_spec = pl.BlockSpec((R, HALF), lambda i, c: (c * NRT + i, 0))
  else:
    out_sds = jax.ShapeDtypeStruct((N, D), jnp.float32)
    out_spec = pl.BlockSpec((R, HALF), lambda i, c: (i, c))
  return pl.pallas_call(
      body,
      grid=(NRT, NC),
      in_specs=[
          pl.BlockSpec((R, HALF), lambda i, c: (i, 0)),
          pl.BlockSpec((R, HALF), lambda i, c: (NRT + i, 0)),
          pl.BlockSpec((R, HALF), lambda i, c: (i, 0)),
          pl.BlockSpec((R, HALF), lambda i, c: (NRT + i, 0)),
          pl.BlockSpec((R, HALF), lambda i, c: (i, 0)),
          pl.BlockSpec((R, HALF), lambda i, c: (NRT + i, 0)),
          pl.BlockSpec((D, HALF), lambda i, c: (0, c)),
          pl.BlockSpec((D, HALF), lambda i, c: (0, c)),
          pl.BlockSpec((1, 1, HALF), lambda i, c: (c, 0, 0)),
      ],
      out_specs=out_spec,
      out_shape=out_sds,
  )


_dense_relu = _dense_build(True, True)
_dense_final = _dense_build(False, False)


@jax.jit
def kernel(x, edge_index, W1l, W1r, b1, W2l, W2r, b2):
  src = edge_index[0].astype(jnp.int32)
  dst = edge_index[1].astype(jnp.int32)
  srcf = jnp.concatenate([src, jnp.zeros((EPAD,), jnp.int32)])
  srcp = jnp.concatenate([srcf, srcf + N])
  dstp = jnp.concatenate([dst, jnp.full((EPAD,), N, jnp.int32)])
  xblk = x.reshape(N, NC, HALF).transpose(1, 0, 2).reshape(NC * N, HALF)
  zblk = jnp.zeros((WA, HALF), jnp.float32)
  onesw = jnp.ones((CHUNK, HALF), jnp.float32)
  b1b = b1.reshape(NC, 1, HALF)
  b2b = b2.reshape(NC, 1, HALF)

  cnt = _sc_cnt(dstp, zblk, onesw)
  agg = _sc_agg(xblk, srcp, dstp, zblk)
  h1 = _dense_relu(agg, agg, xblk, xblk, cnt, cnt, W1l, W1r, b1b)
  agg = _sc_agg(h1, srcp, dstp, zblk)
  h2 = _dense_relu(agg, agg, h1, h1, cnt, cnt, W2l, W2r, b2b)
  agg = _sc_agg(h2, srcp, dstp, zblk)
  return _dense_final(agg, agg, h2, h2, cnt, cnt, W2l, W2r, b2b)


# speedup vs baseline: 3.3560x; 1.0961x over previous
import jax
import jax.numpy as jnp
from jax import lax
from jax.experimental import pallas as pl
from jax.experimental.pallas import tpu as pltpu
from jax.experimental.pallas import tpu_sc as plsc

N = 10000
E = 320000
D = 256
HALF = 128
NC = 2
NS = 16
CHUNK = 128
NBUF = 2
SG = 16
NCHUNK = 160
NSG = NCHUNK // SG
CCHUNK = 80
CPT = NCHUNK * CHUNK
EP = NS * CPT
EPAD = EP - E
ACCN = N + 8
WA = 640
WI = ACCN - (NS - 1) * WA
WO = N - (NS - 1) * WA

_MESH = plsc.VectorSubcoreMesh(core_axis_name="c", subcore_axis_name="s")


def _agg_body(xblk, srcp, dstp, zblk, agg, src_all, dst_all,
              g0, g1, acc, s0, s1):
  c = lax.axis_index("c")
  s = lax.axis_index("s")
  start = pl.multiple_of(s * WA, 8)

  @pl.when(s < NS - 1)
  def _():
    pltpu.sync_copy(zblk, acc.at[pl.ds(start, WA)])

  @pl.when(s == NS - 1)
  def _():
    pltpu.sync_copy(zblk.at[pl.ds(0, WI)], acc.at[pl.ds(start, WI)])

  plsc.subcore_barrier()

  src_base = (c * NS + s) * CPT
  dst_base = s * CPT

  gbufs = (g0, g1)
  sems = (s0, s1)

  def supergroup(jsg, carry):
    sg_off = pl.multiple_of(jsg * (SG * CHUNK), 128)
    pltpu.sync_copy(srcp.at[pl.ds(src_base + sg_off, SG * CHUNK)], src_all)
    pltpu.sync_copy(dstp.at[pl.ds(dst_base + sg_off, SG * CHUNK)], dst_all)

    cps = []
    for b in range(NBUF):
      cps.append(
          pltpu.async_copy(
              xblk.at[src_all.at[pl.ds(b * CHUNK, CHUNK)]], gbufs[b],
              sems[b]))
    for j in range(SG):
      b = j % NBUF
      cps[b].wait()
      pltpu.sync_copy(
          gbufs[b], acc.at[dst_all.at[pl.ds(j * CHUNK, CHUNK)]], add=True)
      nxt = j + NBUF
      if nxt < SG:
        cps[b] = pltpu.async_copy(
            xblk.at[src_all.at[pl.ds(nxt * CHUNK, CHUNK)]], gbufs[b],
            sems[b])
    return carry

  lax.fori_loop(0, NSG, supergroup, 0)

  plsc.subcore_barrier()

  out_start = pl.multiple_of(c * N + s * WA, 8)

  @pl.when(s < NS - 1)
  def _():
    pltpu.sync_copy(acc.at[pl.ds(start, WA)], agg.at[pl.ds(out_start, WA)])

  @pl.when(s == NS - 1)
  def _():
    pltpu.sync_copy(acc.at[pl.ds(start, WO)], agg.at[pl.ds(out_start, WO)])


_sc_agg = pl.kernel(
    _agg_body,
    out_type=jax.ShapeDtypeStruct((NC * N, HALF), jnp.float32),
    mesh=_MESH,
    scratch_types=[
        pltpu.VMEM((SG * CHUNK,), jnp.int32),
        pltpu.VMEM((SG * CHUNK,), jnp.int32),
        pltpu.VMEM((CHUNK, HALF), jnp.float32),
        pltpu.VMEM((CHUNK, HALF), jnp.float32),
        pltpu.VMEM_SHARED((ACCN, HALF), jnp.float32),
        pltpu.SemaphoreType.DMA,
        pltpu.SemaphoreType.DMA,
    ],
)


def _cnt_body(dstp, zblk, onesw, cntb, dst_v, ones_v, acc):
  c = lax.axis_index("c")
  s = lax.axis_index("s")
  start = pl.multiple_of(s * WA, 8)

  @pl.when(s < NS - 1)
  def _():
    pltpu.sync_copy(zblk, acc.at[pl.ds(start, WA)])

  @pl.when(s == NS - 1)
  def _():
    pltpu.sync_copy(zblk.at[pl.ds(0, WI)], acc.at[pl.ds(start, WI)])

  pltpu.sync_copy(onesw, ones_v)
  plsc.subcore_barrier()

  dst_base = (c * NS + s) * CCHUNK * CHUNK

  def chunk(j, carry):
    off = pl.multiple_of(j * CHUNK, 8)
    pltpu.sync_copy(dstp.at[pl.ds(dst_base + off, CHUNK)], dst_v)
    pltpu.sync_copy(ones_v, acc.at[dst_v], add=True)
    return carry

  lax.fori_loop(0, CCHUNK, chunk, 0)

  plsc.subcore_barrier()

  out_start = pl.multiple_of(c * N + s * WA, 8)

  @pl.when(s < NS - 1)
  def _():
    pltpu.sync_copy(acc.at[pl.ds(start, WA)], cntb.at[pl.ds(out_start, WA)])

  @pl.when(s == NS - 1)
  def _():
    pltpu.sync_copy(acc.at[pl.ds(start, WO)], cntb.at[pl.ds(out_start, WO)])


_sc_cnt = pl.kernel(
    _cnt_body,
    out_type=jax.ShapeDtypeStruct((NC * N, HALF), jnp.float32),
    mesh=_MESH,
    scratch_types=[
        pltpu.VMEM((CHUNK,), jnp.int32),
        pltpu.VMEM((CHUNK, HALF), jnp.float32),
        pltpu.VMEM_SHARED((ACCN, HALF), jnp.float32),
    ],
)

R = 1000
NRT = N // R


def _dense_build(do_relu, blocked_out):
  def body(agga, aggb, xa, xb, cnta, cntb, wl, wr, bias, out):
    inv = 1.0 / jnp.maximum(cnta[:, 0:1] + cntb[:, 0:1], 1.0)
    wl_ = wl[...]
    wr_ = wr[...]
    ma = agga[...] * inv
    mb = aggb[...] * inv
    r = (
        jnp.dot(ma, wl_[:HALF], preferred_element_type=jnp.float32)
        + jnp.dot(mb, wl_[HALF:], preferred_element_type=jnp.float32)
        + jnp.dot(xa[...], wr_[:HALF], preferred_element_type=jnp.float32)
        + jnp.dot(xb[...], wr_[HALF:], preferred_element_type=jnp.float32)
        + bias[0]
    )
    if do_relu:
      r = jnp.maximum(r, 0.0)
    out[...] = r

  if blocked_out:
    out_sds = jax.ShapeDtypeStruct((NC * N, HALF), jnp.float32)
    out_spec = pl.BlockSpec((R, HALF), lambda i, c: (c * NRT + i, 0))
  else:
    out_sds = jax.ShapeDtypeStruct((N, D), jnp.float32)
    out_spec = pl.BlockSpec((R, HALF), lambda i, c: (i, c))
  return pl.pallas_call(
      body,
      grid=(NRT, NC),
      in_specs=[
          pl.BlockSpec((R, HALF), lambda i, c: (i, 0)),
          pl.BlockSpec((R, HALF), lambda i, c: (NRT + i, 0)),
          pl.BlockSpec((R, HALF), lambda i, c: (i, 0)),
          pl.BlockSpec((R, HALF), lambda i, c: (NRT + i, 0)),
          pl.BlockSpec((R, HALF), lambda i, c: (i, 0)),
          pl.BlockSpec((R, HALF), lambda i, c: (NRT + i, 0)),
          pl.BlockSpec((D, HALF), lambda i, c: (0, c)),
          pl.BlockSpec((D, HALF), lambda i, c: (0, c)),
          pl.BlockSpec((1, 1, HALF), lambda i, c: (c, 0, 0)),
      ],
      out_specs=out_spec,
      out_shape=out_sds,
  )


_dense_relu = _dense_build(True, True)
_dense_final = _dense_build(False, False)


@jax.jit
def kernel(x, edge_index, W1l, W1r, b1, W2l, W2r, b2):
  src = edge_index[0].astype(jnp.int32)
  dst = edge_index[1].astype(jnp.int32)
  srcf = jnp.concatenate([src, jnp.zeros((EPAD,), jnp.int32)])
  srcp = jnp.concatenate([srcf, srcf + N])
  dstp = jnp.concatenate([dst, jnp.full((EPAD,), N, jnp.int32)])
  xblk = x.reshape(N, NC, HALF).transpose(1, 0, 2).reshape(NC * N, HALF)
  zblk = jnp.zeros((WA, HALF), jnp.float32)
  onesw = jnp.ones((CHUNK, HALF), jnp.float32)
  b1b = b1.reshape(NC, 1, HALF)
  b2b = b2.reshape(NC, 1, HALF)

  cnt = _sc_cnt(dstp, zblk, onesw)
  agg = _sc_agg(xblk, srcp, dstp, zblk)
  h1 = _dense_relu(agg, agg, xblk, xblk, cnt, cnt, W1l, W1r, b1b)
  agg = _sc_agg(h1, srcp, dstp, zblk)
  h2 = _dense_relu(agg, agg, h1, h1, cnt, cnt, W2l, W2r, b2b)
  agg = _sc_agg(h2, srcp, dstp, zblk)
  return _dense_final(agg, agg, h2, h2, cnt, cnt, W2l, W2r, b2b)


# speedup vs baseline: 3.5092x; 1.0456x over previous
import jax
import jax.numpy as jnp
from jax import lax
from jax.experimental import pallas as pl
from jax.experimental.pallas import tpu as pltpu
from jax.experimental.pallas import tpu_sc as plsc

N = 10000
E = 320000
D = 256
HALF = 128
NC = 2
NS = 16
CHUNK = 128
NBUF = 2
SG = 16
NCHUNK = 160
NSG = NCHUNK // SG
CCHUNK = 80
CPT = NCHUNK * CHUNK
EP = NS * CPT
EPAD = EP - E
ACCN = N + 8
WA = 640
WI = ACCN - (NS - 1) * WA
WO = N - (NS - 1) * WA

_MESH = plsc.VectorSubcoreMesh(core_axis_name="c", subcore_axis_name="s")


def _agg_body(xblk, srcp, dstp, zblk, agg, sa0, da0, sa1, da1,
              g0, g1, acc, t0, t1, t2, t3, s0, s1):
  c = lax.axis_index("c")
  s = lax.axis_index("s")
  start = pl.multiple_of(s * WA, 8)

  @pl.when(s < NS - 1)
  def _():
    pltpu.sync_copy(zblk, acc.at[pl.ds(start, WA)])

  @pl.when(s == NS - 1)
  def _():
    pltpu.sync_copy(zblk.at[pl.ds(0, WI)], acc.at[pl.ds(start, WI)])

  plsc.subcore_barrier()

  src_base = (c * NS + s) * CPT
  dst_base = s * CPT

  gbufs = (g0, g1)
  gsems = (s0, s1)
  sbufs = (sa0, sa1)
  dbufs = (da0, da1)
  ssems = ((t0, t1), (t2, t3))
  TOT = NSG * SG

  def sg_stage(jsg):
    p = jsg & 1
    sg_off = jsg * (SG * CHUNK)
    c1 = pltpu.async_copy(
        srcp.at[pl.ds(src_base + sg_off, SG * CHUNK)], sbufs[p], ssems[p][0])
    c2 = pltpu.async_copy(
        dstp.at[pl.ds(dst_base + sg_off, SG * CHUNK)], dbufs[p], ssems[p][1])
    return (c1, c2)

  def fire(k):
    jsg, jj = divmod(k, SG)
    p = jsg & 1
    b = k % NBUF
    return pltpu.async_copy(
        xblk.at[sbufs[p].at[pl.ds(jj * CHUNK, CHUNK)]], gbufs[b], gsems[b])

  stage_h = [None] * NSG
  staged = [False] * NSG
  stage_h[0] = sg_stage(0)
  stage_h[1] = sg_stage(1)

  def ensure(jsg):
    if not staged[jsg]:
      stage_h[jsg][0].wait()
      stage_h[jsg][1].wait()
      staged[jsg] = True

  ensure(0)
  pend = [fire(b) for b in range(NBUF)]

  for k in range(TOT):
    jsg, jj = divmod(k, SG)
    if jj == 0 and 1 <= jsg and jsg + 1 < NSG:
      stage_h[jsg + 1] = sg_stage(jsg + 1)
    b = k % NBUF
    pend[b].wait()
    p = jsg & 1
    pltpu.sync_copy(
        gbufs[b], acc.at[dbufs[p].at[pl.ds(jj * CHUNK, CHUNK)]], add=True)
    nk = k + NBUF
    if nk < TOT:
      nsg_, njj = divmod(nk, SG)
      if njj == 0:
        ensure(nsg_)
      pend[b] = fire(nk)

  plsc.subcore_barrier()

  out_start = pl.multiple_of(c * N + s * WA, 8)

  @pl.when(s < NS - 1)
  def _():
    pltpu.sync_copy(acc.at[pl.ds(start, WA)], agg.at[pl.ds(out_start, WA)])

  @pl.when(s == NS - 1)
  def _():
    pltpu.sync_copy(acc.at[pl.ds(start, WO)], agg.at[pl.ds(out_start, WO)])


_sc_agg = pl.kernel(
    _agg_body,
    out_type=jax.ShapeDtypeStruct((NC * N, HALF), jnp.float32),
    mesh=_MESH,
    scratch_types=[
        pltpu.VMEM((SG * CHUNK,), jnp.int32),
        pltpu.VMEM((SG * CHUNK,), jnp.int32),
        pltpu.VMEM((SG * CHUNK,), jnp.int32),
        pltpu.VMEM((SG * CHUNK,), jnp.int32),
        pltpu.VMEM((CHUNK, HALF), jnp.float32),
        pltpu.VMEM((CHUNK, HALF), jnp.float32),
        pltpu.VMEM_SHARED((ACCN, HALF), jnp.float32),
        pltpu.SemaphoreType.DMA,
        pltpu.SemaphoreType.DMA,
        pltpu.SemaphoreType.DMA,
        pltpu.SemaphoreType.DMA,
        pltpu.SemaphoreType.DMA,
        pltpu.SemaphoreType.DMA,
    ],
)


def _cnt_body(dstp, zblk, onesw, cntb, dst_v, ones_v, acc):
  c = lax.axis_index("c")
  s = lax.axis_index("s")
  start = pl.multiple_of(s * WA, 8)

  @pl.when(s < NS - 1)
  def _():
    pltpu.sync_copy(zblk, acc.at[pl.ds(start, WA)])

  @pl.when(s == NS - 1)
  def _():
    pltpu.sync_copy(zblk.at[pl.ds(0, WI)], acc.at[pl.ds(start, WI)])

  pltpu.sync_copy(onesw, ones_v)
  plsc.subcore_barrier()

  dst_base = (c * NS + s) * CCHUNK * CHUNK

  def chunk(j, carry):
    off = pl.multiple_of(j * CHUNK, 8)
    pltpu.sync_copy(dstp.at[pl.ds(dst_base + off, CHUNK)], dst_v)
    pltpu.sync_copy(ones_v, acc.at[dst_v], add=True)
    return carry

  lax.fori_loop(0, CCHUNK, chunk, 0)

  plsc.subcore_barrier()

  out_start = pl.multiple_of(c * N + s * WA, 8)

  @pl.when(s < NS - 1)
  def _():
    pltpu.sync_copy(acc.at[pl.ds(start, WA)], cntb.at[pl.ds(out_start, WA)])

  @pl.when(s == NS - 1)
  def _():
    pltpu.sync_copy(acc.at[pl.ds(start, WO)], cntb.at[pl.ds(out_start, WO)])


_sc_cnt = pl.kernel(
    _cnt_body,
    out_type=jax.ShapeDtypeStruct((NC * N, HALF), jnp.float32),
    mesh=_MESH,
    scratch_types=[
        pltpu.VMEM((CHUNK,), jnp.int32),
        pltpu.VMEM((CHUNK, HALF), jnp.float32),
        pltpu.VMEM_SHARED((ACCN, HALF), jnp.float32),
    ],
)

R = 1000
NRT = N // R


def _dense_build(do_relu, blocked_out):
  def body(agga, aggb, xa, xb, cnta, cntb, wl, wr, bias, out):
    inv = 1.0 / jnp.maximum(cnta[:, 0:1] + cntb[:, 0:1], 1.0)
    wl_ = wl[...]
    wr_ = wr[...]
    ma = agga[...] * inv
    mb = aggb[...] * inv
    r = (
        jnp.dot(ma, wl_[:HALF], preferred_element_type=jnp.float32)
        + jnp.dot(mb, wl_[HALF:], preferred_element_type=jnp.float32)
        + jnp.dot(xa[...], wr_[:HALF], preferred_element_type=jnp.float32)
        + jnp.dot(xb[...], wr_[HALF:], preferred_element_type=jnp.float32)
        + bias[0]
    )
    if do_relu:
      r = jnp.maximum(r, 0.0)
    out[...] = r

  if blocked_out:
    out_sds = jax.ShapeDtypeStruct((NC * N, HALF), jnp.float32)
    out_spec = pl.BlockSpec((R, HALF), lambda i, c: (c * NRT + i, 0))
  else:
    out_sds = jax.ShapeDtypeStruct((N, D), jnp.float32)
    out_spec = pl.BlockSpec((R, HALF), lambda i, c: (i, c))
  return pl.pallas_call(
      body,
      grid=(NRT, NC),
      in_specs=[
          pl.BlockSpec((R, HALF), lambda i, c: (i, 0)),
          pl.BlockSpec((R, HALF), lambda i, c: (NRT + i, 0)),
          pl.BlockSpec((R, HALF), lambda i, c: (i, 0)),
          pl.BlockSpec((R, HALF), lambda i, c: (NRT + i, 0)),
          pl.BlockSpec((R, HALF), lambda i, c: (i, 0)),
          pl.BlockSpec((R, HALF), lambda i, c: (NRT + i, 0)),
          pl.BlockSpec((D, HALF), lambda i, c: (0, c)),
          pl.BlockSpec((D, HALF), lambda i, c: (0, c)),
          pl.BlockSpec((1, 1, HALF), lambda i, c: (c, 0, 0)),
      ],
      out_specs=out_spec,
      out_shape=out_sds,
  )


_dense_relu = _dense_build(True, True)
_dense_final = _dense_build(False, False)


@jax.jit
def kernel(x, edge_index, W1l, W1r, b1, W2l, W2r, b2):
  src = edge_index[0].astype(jnp.int32)
  dst = edge_index[1].astype(jnp.int32)
  srcf = jnp.concatenate([src, jnp.zeros((EPAD,), jnp.int32)])
  srcp = jnp.concatenate([srcf, srcf + N])
  dstp = jnp.concatenate([dst, jnp.full((EPAD,), N, jnp.int32)])
  xblk = x.reshape(N, NC, HALF).transpose(1, 0, 2).reshape(NC * N, HALF)
  zblk = jnp.zeros((WA, HALF), jnp.float32)
  onesw = jnp.ones((CHUNK, HALF), jnp.float32)
  b1b = b1.reshape(NC, 1, HALF)
  b2b = b2.reshape(NC, 1, HALF)

  cnt = _sc_cnt(dstp, zblk, onesw)
  agg = _sc_agg(xblk, srcp, dstp, zblk)
  h1 = _dense_relu(agg, agg, xblk, xblk, cnt, cnt, W1l, W1r, b1b)
  agg = _sc_agg(h1, srcp, dstp, zblk)
  h2 = _dense_relu(agg, agg, h1, h1, cnt, cnt, W2l, W2r, b2b)
  agg = _sc_agg(h2, srcp, dstp, zblk)
  return _dense_final(agg, agg, h2, h2, cnt, cnt, W2l, W2r, b2b)


# speedup vs baseline: 3.5680x; 1.0167x over previous
import jax
import jax.numpy as jnp
from jax import lax
from jax.experimental import pallas as pl
from jax.experimental.pallas import tpu as pltpu
from jax.experimental.pallas import tpu_sc as plsc

N = 10000
E = 320000
D = 256
HALF = 128
NC = 2
NS = 16
CHUNK = 128
NBUF = 2
SG = 16
NCHUNK = 160
NSG = NCHUNK // SG
CCHUNK = 80
CPT = NCHUNK * CHUNK
EP = NS * CPT
EPAD = EP - E
ACCN = N + 8
WA = 640
WI = ACCN - (NS - 1) * WA
WO = N - (NS - 1) * WA

_MESH = plsc.VectorSubcoreMesh(core_axis_name="c", subcore_axis_name="s")


def _agg_body(xblk, srcp, dstp, zblk, agg, sa0, da0, sa1, da1,
              g0, g1, acc, t0, t1, t2, t3, s0, s1):
  c = lax.axis_index("c")
  s = lax.axis_index("s")
  start = pl.multiple_of(s * WA, 8)

  @pl.when(s < NS - 1)
  def _():
    pltpu.sync_copy(zblk, acc.at[pl.ds(start, WA)])

  @pl.when(s == NS - 1)
  def _():
    pltpu.sync_copy(zblk.at[pl.ds(0, WI)], acc.at[pl.ds(start, WI)])

  plsc.subcore_barrier()

  src_base = (c * NS + s) * CPT
  dst_base = s * CPT

  gbufs = (g0, g1)
  gsems = (s0, s1)
  sbufs = (sa0, sa1)
  dbufs = (da0, da1)
  ssems = ((t0, t1), (t2, t3))
  TOT = NSG * SG

  def sg_stage(jsg):
    p = jsg & 1
    sg_off = jsg * (SG * CHUNK)
    c1 = pltpu.async_copy(
        srcp.at[pl.ds(src_base + sg_off, SG * CHUNK)], sbufs[p], ssems[p][0])
    c2 = pltpu.async_copy(
        dstp.at[pl.ds(dst_base + sg_off, SG * CHUNK)], dbufs[p], ssems[p][1])
    return (c1, c2)

  def fire(k):
    jsg, jj = divmod(k, SG)
    p = jsg & 1
    b = k % NBUF
    return pltpu.async_copy(
        xblk.at[sbufs[p].at[pl.ds(jj * CHUNK, CHUNK)]], gbufs[b], gsems[b])

  stage_h = [None] * NSG
  staged = [False] * NSG
  stage_h[0] = sg_stage(0)
  stage_h[1] = sg_stage(1)

  def ensure(jsg):
    if not staged[jsg]:
      stage_h[jsg][0].wait()
      stage_h[jsg][1].wait()
      staged[jsg] = True

  ensure(0)
  pend = [fire(b) for b in range(NBUF)]

  for k in range(TOT):
    jsg, jj = divmod(k, SG)
    if jj == 0 and 1 <= jsg and jsg + 1 < NSG:
      stage_h[jsg + 1] = sg_stage(jsg + 1)
    b = k % NBUF
    pend[b].wait()
    p = jsg & 1
    pltpu.sync_copy(
        gbufs[b], acc.at[dbufs[p].at[pl.ds(jj * CHUNK, CHUNK)]], add=True)
    nk = k + NBUF
    if nk < TOT:
      nsg_, njj = divmod(nk, SG)
      if njj == 0:
        ensure(nsg_)
      pend[b] = fire(nk)

  plsc.subcore_barrier()

  out_start = pl.multiple_of(c * N + s * WA, 8)

  @pl.when(s < NS - 1)
  def _():
    pltpu.sync_copy(acc.at[pl.ds(start, WA)], agg.at[pl.ds(out_start, WA)])

  @pl.when(s == NS - 1)
  def _():
    pltpu.sync_copy(acc.at[pl.ds(start, WO)], agg.at[pl.ds(out_start, WO)])


_sc_agg = pl.kernel(
    _agg_body,
    out_type=jax.ShapeDtypeStruct((NC * N, HALF), jnp.float32),
    mesh=_MESH,
    scratch_types=[
        pltpu.VMEM((SG * CHUNK,), jnp.int32),
        pltpu.VMEM((SG * CHUNK,), jnp.int32),
        pltpu.VMEM((SG * CHUNK,), jnp.int32),
        pltpu.VMEM((SG * CHUNK,), jnp.int32),
        pltpu.VMEM((CHUNK, HALF), jnp.float32),
        pltpu.VMEM((CHUNK, HALF), jnp.float32),
        pltpu.VMEM_SHARED((ACCN, HALF), jnp.float32),
        pltpu.SemaphoreType.DMA,
        pltpu.SemaphoreType.DMA,
        pltpu.SemaphoreType.DMA,
        pltpu.SemaphoreType.DMA,
        pltpu.SemaphoreType.DMA,
        pltpu.SemaphoreType.DMA,
    ],
)


def _cnt_body(dstp, zblk, onesw, cntb, dst_v, ones_v, acc):
  c = lax.axis_index("c")
  s = lax.axis_index("s")
  start = pl.multiple_of(s * WA, 8)

  @pl.when(s < NS - 1)
  def _():
    pltpu.sync_copy(zblk, acc.at[pl.ds(start, WA)])

  @pl.when(s == NS - 1)
  def _():
    pltpu.sync_copy(zblk.at[pl.ds(0, WI)], acc.at[pl.ds(start, WI)])

  pltpu.sync_copy(onesw, ones_v)
  plsc.subcore_barrier()

  dst_base = (c * NS + s) * CCHUNK * CHUNK

  def chunk(j, carry):
    off = pl.multiple_of(j * CHUNK, 8)
    pltpu.sync_copy(dstp.at[pl.ds(dst_base + off, CHUNK)], dst_v)
    pltpu.sync_copy(ones_v, acc.at[dst_v], add=True)
    return carry

  lax.fori_loop(0, CCHUNK, chunk, 0)

  plsc.subcore_barrier()

  out_start = pl.multiple_of(c * N + s * WA, 8)

  @pl.when(s < NS - 1)
  def _():
    pltpu.sync_copy(acc.at[pl.ds(start, WA)], cntb.at[pl.ds(out_start, WA)])

  @pl.when(s == NS - 1)
  def _():
    pltpu.sync_copy(acc.at[pl.ds(start, WO)], cntb.at[pl.ds(out_start, WO)])


_sc_cnt = pl.kernel(
    _cnt_body,
    out_type=jax.ShapeDtypeStruct((NC * N, HALF), jnp.float32),
    mesh=_MESH,
    scratch_types=[
        pltpu.VMEM((CHUNK,), jnp.int32),
        pltpu.VMEM((CHUNK, HALF), jnp.float32),
        pltpu.VMEM_SHARED((ACCN, HALF), jnp.float32),
    ],
)

R = 1000
NRT = N // R


def _self_build():
  def body(xa, xb, wr, bias, out):
    wr_ = wr[...]
    out[...] = (
        jnp.dot(xa[...], wr_[:HALF], preferred_element_type=jnp.float32)
        + jnp.dot(xb[...], wr_[HALF:], preferred_element_type=jnp.float32)
        + bias[0]
    )

  return pl.pallas_call(
      body,
      grid=(NRT, NC),
      in_specs=[
          pl.BlockSpec((R, HALF), lambda i, c: (i, 0)),
          pl.BlockSpec((R, HALF), lambda i, c: (NRT + i, 0)),
          pl.BlockSpec((D, HALF), lambda i, c: (0, c)),
          pl.BlockSpec((1, 1, HALF), lambda i, c: (c, 0, 0)),
      ],
      out_specs=pl.BlockSpec((R, HALF), lambda i, c: (c * NRT + i, 0)),
      out_shape=jax.ShapeDtypeStruct((NC * N, HALF), jnp.float32),
  )


def _comb_build(do_relu, blocked_out):
  def body(agga, aggb, cnta, cntb, wl, sblk, out):
    inv = 1.0 / jnp.maximum(cnta[:, 0:1] + cntb[:, 0:1], 1.0)
    wl_ = wl[...]
    r = (
        jnp.dot(agga[...] * inv, wl_[:HALF],
                preferred_element_type=jnp.float32)
        + jnp.dot(aggb[...] * inv, wl_[HALF:],
                  preferred_element_type=jnp.float32)
        + sblk[...]
    )
    if do_relu:
      r = jnp.maximum(r, 0.0)
    out[...] = r

  if blocked_out:
    out_sds = jax.ShapeDtypeStruct((NC * N, HALF), jnp.float32)
    out_spec = pl.BlockSpec((R, HALF), lambda i, c: (c * NRT + i, 0))
  else:
    out_sds = jax.ShapeDtypeStruct((N, D), jnp.float32)
    out_spec = pl.BlockSpec((R, HALF), lambda i, c: (i, c))
  return pl.pallas_call(
      body,
      grid=(NRT, NC),
      in_specs=[
          pl.BlockSpec((R, HALF), lambda i, c: (i, 0)),
          pl.BlockSpec((R, HALF), lambda i, c: (NRT + i, 0)),
          pl.BlockSpec((R, HALF), lambda i, c: (i, 0)),
          pl.BlockSpec((R, HALF), lambda i, c: (NRT + i, 0)),
          pl.BlockSpec((D, HALF), lambda i, c: (0, c)),
          pl.BlockSpec((R, HALF), lambda i, c: (c * NRT + i, 0)),
      ],
      out_specs=out_spec,
      out_shape=out_sds,
  )


_dense_self = _self_build()
_comb_relu = _comb_build(True, True)
_comb_final = _comb_build(False, False)


@jax.jit
def kernel(x, edge_index, W1l, W1r, b1, W2l, W2r, b2):
  src = edge_index[0].astype(jnp.int32)
  dst = edge_index[1].astype(jnp.int32)
  srcf = jnp.concatenate([src, jnp.zeros((EPAD,), jnp.int32)])
  srcp = jnp.concatenate([srcf, srcf + N])
  dstp = jnp.concatenate([dst, jnp.full((EPAD,), N, jnp.int32)])
  xblk = x.reshape(N, NC, HALF).transpose(1, 0, 2).reshape(NC * N, HALF)
  zblk = jnp.zeros((WA, HALF), jnp.float32)
  onesw = jnp.ones((CHUNK, HALF), jnp.float32)
  b1b = b1.reshape(NC, 1, HALF)
  b2b = b2.reshape(NC, 1, HALF)

  cnt = _sc_cnt(dstp, zblk, onesw)
  s1 = _dense_self(xblk, xblk, W1r, b1b)
  agg = _sc_agg(xblk, srcp, dstp, zblk)
  h1 = _comb_relu(agg, agg, cnt, cnt, W1l, s1)
  s2 = _dense_self(h1, h1, W2r, b2b)
  agg = _sc_agg(h1, srcp, dstp, zblk)
  h2 = _comb_relu(agg, agg, cnt, cnt, W2l, s2)
  s3 = _dense_self(h2, h2, W2r, b2b)
  agg = _sc_agg(h2, srcp, dstp, zblk)
  return _comb_final(agg, agg, cnt, cnt, W2l, s3)
